# baseline (speedup 1.0000x reference)
import functools

import jax
import jax.numpy as jnp
from jax import lax
from jax.experimental import pallas as pl
from jax.experimental.pallas import tpu as pltpu
from jax.experimental.pallas import tpu_sc as plsc

NB_CODE = 8192
CODE_DIM = 64
B, N, Q = 32, 2048, 4

NC, NS = 2, 16
NW = NC * NS
NT = N // 128
BLOCKS = NT * Q
PITCH = 128


def _vq_gather(table, idx_p):
    mesh = plsc.VectorSubcoreMesh(core_axis_name="c", subcore_axis_name="s")

    @functools.partial(
        pl.kernel,
        mesh=mesh,
        out_type=jax.ShapeDtypeStruct((B, Q, 8, NT, 8, 128), jnp.float32),
        compiler_params=pltpu.CompilerParams(use_tc_tiling_on_sc=False,
                                             needs_layout_passes=False),
        scratch_types=[
            pltpu.VMEM((NT, Q, 128), jnp.int32),
            pltpu.VMEM((128, CODE_DIM), jnp.float32),
            pltpu.VMEM((128, CODE_DIM), jnp.float32),
            pltpu.VMEM((8, 8, PITCH), jnp.float32),
            pltpu.VMEM((8, 8, PITCH), jnp.float32),
            pltpu.SemaphoreType.DMA,
            pltpu.SemaphoreType.DMA,
            pltpu.SemaphoreType.DMA,
            pltpu.SemaphoreType.DMA,
        ],
    )
    def k(table_hbm, idx_hbm, out_hbm, idx_v, rows_a, rows_b,
          tiles_a, tiles_b, gs_a, gs_b, os_a, os_b):
        wid = lax.axis_index("s") * NC + lax.axis_index("c")
        pltpu.sync_copy(idx_hbm.at[wid], idx_v)

        lane = jax.lax.iota(jnp.int32, 16)
        dr_vec = lane & 7
        dt_base = jax.lax.shift_right_logical(lane, jnp.int32(3))
        dt_vecs = [dt_base + 2 * c for c in range(4)]
        zero16 = lane & 0

        def fire_gather(j, rows, gsem):
            nt = j // Q
            q = j % Q
            pltpu.async_copy(table_hbm.at[idx_v.at[nt, q]], rows, gsem)

        def drain_gather(j, rows, gsem):
            nt = j // Q
            q = j % Q
            pltpu.make_async_copy(table_hbm.at[idx_v.at[nt, q]], rows,
                                  gsem).wait()

        def fire_out(j, tiles, osem):
            nt = j // Q
            q = j % Q
            pltpu.async_copy(tiles.at[:, :, pl.ds(0, 128)],
                             out_hbm.at[wid, q, :, nt], osem)

        def wait_out(j, tiles, osem):
            nt = j // Q
            q = j % Q
            pltpu.make_async_copy(tiles.at[:, :, pl.ds(0, 128)],
                                  out_hbm.at[wid, q, :, nt], osem).wait()

        def transpose(rows, tiles):
            def tbody(i, carry):
                tb = i * 16
                for u in range(16):
                    tsplat = zero16 + (tb + u)
                    for c in range(4):
                        v = rows[tb + u, pl.ds(16 * c, 16)]
                        plsc.store_scatter(tiles, [dt_vecs[c], dr_vec,
                                                   tsplat], v)
                return carry

            lax.fori_loop(0, 8, tbody, 0)

        fire_gather(0, rows_a, gs_a)

        def body(i, carry):
            ja = 2 * i
            jb = ja + 1
            fire_gather(jb, rows_b, gs_b)
            drain_gather(ja, rows_a, gs_a)

            @pl.when(i > 0)
            def _():
                wait_out(ja - 2, tiles_a, os_a)
            transpose(rows_a, tiles_a)
            fire_out(ja, tiles_a, os_a)

            @pl.when(i < BLOCKS // 2 - 1)
            def _():
                fire_gather(ja + 2, rows_a, gs_a)
            drain_gather(jb, rows_b, gs_b)

            @pl.when(i > 0)
            def _():
                wait_out(jb - 2, tiles_b, os_b)
            transpose(rows_b, tiles_b)
            fire_out(jb, tiles_b, os_b)
            return carry

        lax.fori_loop(0, BLOCKS // 2, body, 0)
        wait_out(BLOCKS - 2, tiles_a, os_a)
        wait_out(BLOCKS - 1, tiles_b, os_b)

    return k(table, idx_p)


def kernel(idx, codebook, padding_embedding):
    row = jax.lax.broadcasted_iota(jnp.int32, (NB_CODE, 1), 0)
    table = jnp.where(row == 0, padding_embedding.reshape(1, CODE_DIM),
                      codebook)
    idx_p = jnp.swapaxes(idx.reshape(B, NT, 128, Q), 2, 3)
    out6 = _vq_gather(table, idx_p)
    return jnp.transpose(out6, (0, 3, 5, 1, 2, 4)).reshape(B, N, Q, CODE_DIM)


# speedup vs baseline: 3.0225x; 3.0225x over previous
import functools

import jax
import jax.numpy as jnp
from jax import lax
from jax.experimental import pallas as pl
from jax.experimental.pallas import tpu as pltpu
from jax.experimental.pallas import tpu_sc as plsc

NB_CODE = 8192
CODE_DIM = 64
B, N, Q = 32, 2048, 4

NC, NS = 2, 16
NW = NC * NS
NT = N // 128
BLOCKS = NT * Q
PITCH = 129


def _vq_gather(table, idx_p):
    mesh = plsc.VectorSubcoreMesh(core_axis_name="c", subcore_axis_name="s")

    @functools.partial(
        pl.kernel,
        mesh=mesh,
        out_type=jax.ShapeDtypeStruct((B, Q, 8, NT, 8, 128), jnp.float32),
        compiler_params=pltpu.CompilerParams(use_tc_tiling_on_sc=False,
                                             needs_layout_passes=False),
        scratch_types=[
            pltpu.VMEM((NT, Q, 128), jnp.int32),
            pltpu.VMEM((128, CODE_DIM), jnp.float32),
            pltpu.VMEM((128, CODE_DIM), jnp.float32),
            pltpu.VMEM((8, 8, PITCH), jnp.float32),
            pltpu.VMEM((8, 8, PITCH), jnp.float32),
            pltpu.SemaphoreType.DMA,
            pltpu.SemaphoreType.DMA,
            pltpu.SemaphoreType.DMA,
            pltpu.SemaphoreType.DMA,
        ],
    )
    def k(table_hbm, idx_hbm, out_hbm, idx_v, rows_a, rows_b,
          tiles_a, tiles_b, gs_a, gs_b, os_a, os_b):
        wid = lax.axis_index("s") * NC + lax.axis_index("c")
        pltpu.sync_copy(idx_hbm.at[wid], idx_v)

        lane = jax.lax.iota(jnp.int32, 16)
        dr_vec = lane & 7
        dt_base = jax.lax.shift_right_logical(lane, jnp.int32(3))
        dt_vecs = [dt_base + 2 * c for c in range(4)]
        zero16 = lane & 0

        def fire_gather(j, rows, gsem):
            nt = j // Q
            q = j % Q
            pltpu.async_copy(table_hbm.at[idx_v.at[nt, q]], rows, gsem)

        def drain_gather(j, rows, gsem):
            nt = j // Q
            q = j % Q
            pltpu.make_async_copy(table_hbm.at[idx_v.at[nt, q]], rows,
                                  gsem).wait()

        def fire_out(j, tiles, osem):
            nt = j // Q
            q = j % Q
            pltpu.async_copy(tiles.at[:, :, pl.ds(0, 128)],
                             out_hbm.at[wid, q, :, nt], osem)

        def wait_out(j, tiles, osem):
            nt = j // Q
            q = j % Q
            pltpu.make_async_copy(tiles.at[:, :, pl.ds(0, 128)],
                                  out_hbm.at[wid, q, :, nt], osem).wait()

        def transpose(rows, tiles):
            def tbody(i, carry):
                tb = i * 16
                for g in range(4):
                    vs = []
                    for u in range(4):
                        t = tb + g * 4 + u
                        for c in range(4):
                            vs.append((t, c, rows[t, pl.ds(16 * c, 16)]))
                    for t, c, v in vs:
                        plsc.store_scatter(tiles, [dt_vecs[c], dr_vec,
                                                   zero16 + t], v)
                return carry

            lax.fori_loop(0, 8, tbody, 0)

        fire_gather(0, rows_a, gs_a)

        def body(i, carry):
            ja = 2 * i
            jb = ja + 1
            fire_gather(jb, rows_b, gs_b)
            drain_gather(ja, rows_a, gs_a)

            @pl.when(i > 0)
            def _():
                wait_out(ja - 2, tiles_a, os_a)
            transpose(rows_a, tiles_a)
            fire_out(ja, tiles_a, os_a)

            @pl.when(i < BLOCKS // 2 - 1)
            def _():
                fire_gather(ja + 2, rows_a, gs_a)
            drain_gather(jb, rows_b, gs_b)

            @pl.when(i > 0)
            def _():
                wait_out(jb - 2, tiles_b, os_b)
            transpose(rows_b, tiles_b)
            fire_out(jb, tiles_b, os_b)
            return carry

        lax.fori_loop(0, BLOCKS // 2, body, 0)
        wait_out(BLOCKS - 2, tiles_a, os_a)
        wait_out(BLOCKS - 1, tiles_b, os_b)

    return k(table, idx_p)


def kernel(idx, codebook, padding_embedding):
    row = jax.lax.broadcasted_iota(jnp.int32, (NB_CODE, 1), 0)
    table = jnp.where(row == 0, padding_embedding.reshape(1, CODE_DIM),
                      codebook)
    idx_p = jnp.swapaxes(idx.reshape(B, NT, 128, Q), 2, 3)
    out6 = _vq_gather(table, idx_p)
    return jnp.transpose(out6, (0, 3, 5, 1, 2, 4)).reshape(B, N, Q, CODE_DIM)
